# baseline (speedup 1.0000x reference)
import functools

import jax
import jax.numpy as jnp
from jax import lax
from jax.experimental import pallas as pl
from jax.experimental.pallas import tpu as pltpu
from jax.experimental.pallas import tpu_sc as plsc

D_MODEL = 128
NUM_WORKERS = 32
CHUNK = 400


def _sc_gather(idx_flat, table, n_total, n_rows):
    n_per_w = n_total // NUM_WORKERS
    steps = n_per_w // CHUNK
    mesh = plsc.VectorSubcoreMesh(core_axis_name="c", subcore_axis_name="s")

    @functools.partial(
        pl.kernel,
        mesh=mesh,
        out_type=jax.ShapeDtypeStruct((n_total, D_MODEL), jnp.float32),
        scratch_types=[
            pltpu.VMEM((n_per_w,), jnp.int32),
            pltpu.VMEM_SHARED((n_rows, D_MODEL), jnp.float32),
            [pltpu.VMEM((CHUNK, D_MODEL), jnp.float32)] * 2,
            [pltpu.SemaphoreType.DMA] * 2,
            [pltpu.SemaphoreType.DMA] * 2,
        ],
    )
    def k(idx_hbm, table_hbm, out_hbm, idx_v, table_v, rows, gsem, wsem):
        sid = lax.axis_index("s")
        wid = sid * 2 + lax.axis_index("c")
        base = wid * n_per_w
        pltpu.sync_copy(idx_hbm.at[pl.ds(base, n_per_w)], idx_v)

        @pl.when(sid == 0)
        def _():
            pltpu.sync_copy(table_hbm, table_v)

        plsc.subcore_barrier()

        def gather(i, b):
            return pltpu.make_async_copy(
                table_v.at[idx_v.at[pl.ds(i * CHUNK, CHUNK)]], rows[b], gsem[b]
            )

        def write(i, b):
            return pltpu.make_async_copy(
                rows[b], out_hbm.at[pl.ds(base + i * CHUNK, CHUNK)], wsem[b]
            )

        gather(0, 0).start()
        gather(0, 0).wait()
        write(0, 0).start()
        gather(1, 1).start()

        def body(grp, carry):
            for bb in range(2):
                i = grp * 2 + 1 + bb
                b = (1 + bb) % 2
                gather(i, b).wait()
                write(i, b).start()
                write(i - 1, 1 - b).wait()
                gather(i + 1, 1 - b).start()
            return carry

        lax.fori_loop(0, (steps - 2) // 2, body, 0)

        i = steps - 1
        gather(i, 1).wait()
        write(i, 1).start()
        write(i - 1, 0).wait()
        write(i, 1).wait()

    return k(idx_flat, table)


def kernel(cumulative_positions, position_embeddings):
    b, s = cumulative_positions.shape
    n_total = b * s
    n_rows = position_embeddings.shape[0]
    idx_flat = cumulative_positions.reshape(n_total).astype(jnp.int32)
    out = _sc_gather(idx_flat, position_embeddings, n_total, n_rows)
    return out.reshape(b, s, D_MODEL)


# speedup vs baseline: 1.0391x; 1.0391x over previous
import functools

import jax
import jax.numpy as jnp
from jax import lax
from jax.experimental import pallas as pl
from jax.experimental.pallas import tpu as pltpu
from jax.experimental.pallas import tpu_sc as plsc

D_MODEL = 128
NUM_WORKERS = 32
CHUNK = 80
NBUF = 8
GA = 4


def _sc_gather(idx_flat, table, n_total, n_rows):
    n_per_w = n_total // NUM_WORKERS
    steps = n_per_w // CHUNK
    n_grps = steps // NBUF
    mesh = plsc.VectorSubcoreMesh(core_axis_name="c", subcore_axis_name="s")

    @functools.partial(
        pl.kernel,
        mesh=mesh,
        out_type=jax.ShapeDtypeStruct((n_total, D_MODEL), jnp.float32),
        scratch_types=[
            pltpu.VMEM((n_per_w,), jnp.int32),
            pltpu.VMEM_SHARED((n_rows, D_MODEL), jnp.float32),
            [pltpu.VMEM((CHUNK, D_MODEL), jnp.float32)] * NBUF,
            [pltpu.SemaphoreType.DMA] * NBUF,
            [pltpu.SemaphoreType.DMA] * NBUF,
        ],
    )
    def k(idx_hbm, table_hbm, out_hbm, idx_v, table_v, rows, gsem, wsem):
        sid = lax.axis_index("s")
        wid = sid * 2 + lax.axis_index("c")
        base = wid * n_per_w
        pltpu.sync_copy(idx_hbm.at[pl.ds(base, n_per_w)], idx_v)

        @pl.when(sid == 0)
        def _():
            pltpu.sync_copy(table_hbm, table_v)

        plsc.subcore_barrier()

        def gather(i, b):
            return pltpu.make_async_copy(
                table_v.at[idx_v.at[pl.ds(i * CHUNK, CHUNK)]], rows[b], gsem[b]
            )

        def write(i, b):
            return pltpu.make_async_copy(
                rows[b], out_hbm.at[pl.ds(base + i * CHUNK, CHUNK)], wsem[b]
            )

        for b in range(GA):
            gather(b, b).start()

        for bb in range(NBUF):
            gather(bb, bb).wait()
            write(bb, bb).start()
            if bb + GA < NBUF:
                gather(bb + GA, bb + GA).start()
            else:
                write(bb + GA - NBUF, (bb + GA) % NBUF).wait()
                gather(bb + GA, (bb + GA) % NBUF).start()

        def body(grp, carry):
            for bb in range(NBUF):
                i = grp * NBUF + bb
                gather(i, bb).wait()
                write(i, bb).start()
                write(i - GA, (bb + GA) % NBUF).wait()
                gather(i + GA, (bb + GA) % NBUF).start()
            return carry

        lax.fori_loop(1, n_grps - 1, body, 0)

        for bb in range(NBUF):
            i = (n_grps - 1) * NBUF + bb
            gather(i, bb).wait()
            write(i, bb).start()
            if bb < GA:
                write(i - GA, (bb + GA) % NBUF).wait()
                gather(i + GA, (bb + GA) % NBUF).start()
        for bb in range(NBUF):
            write((n_grps - 1) * NBUF + bb, bb).wait()

    return k(idx_flat, table)


def kernel(cumulative_positions, position_embeddings):
    b, s = cumulative_positions.shape
    n_total = b * s
    n_rows = position_embeddings.shape[0]
    idx_flat = cumulative_positions.reshape(n_total).astype(jnp.int32)
    out = _sc_gather(idx_flat, position_embeddings, n_total, n_rows)
    return out.reshape(b, s, D_MODEL)


# speedup vs baseline: 1.0424x; 1.0032x over previous
import functools

import jax
import jax.numpy as jnp
from jax import lax
from jax.experimental import pallas as pl
from jax.experimental.pallas import tpu as pltpu
from jax.experimental.pallas import tpu_sc as plsc

D_MODEL = 128
NUM_WORKERS = 32
CHUNK = 64
NBUF = 10
GA = 5


def _sc_gather(idx_flat, table, n_total, n_rows):
    n_per_w = n_total // NUM_WORKERS
    steps = n_per_w // CHUNK
    n_grps = steps // NBUF
    mesh = plsc.VectorSubcoreMesh(core_axis_name="c", subcore_axis_name="s")

    @functools.partial(
        pl.kernel,
        mesh=mesh,
        out_type=jax.ShapeDtypeStruct((n_total, D_MODEL), jnp.float32),
        scratch_types=[
            pltpu.VMEM((n_per_w,), jnp.int32),
            pltpu.VMEM_SHARED((n_rows, D_MODEL), jnp.float32),
            [pltpu.VMEM((CHUNK, D_MODEL), jnp.float32)] * NBUF,
            [pltpu.SemaphoreType.DMA] * NBUF,
            [pltpu.SemaphoreType.DMA] * NBUF,
        ],
    )
    def k(idx_hbm, table_hbm, out_hbm, idx_v, table_v, rows, gsem, wsem):
        sid = lax.axis_index("s")
        wid = sid * 2 + lax.axis_index("c")
        base = wid * n_per_w
        pltpu.sync_copy(idx_hbm.at[pl.ds(base, n_per_w)], idx_v)

        @pl.when(sid == 0)
        def _():
            pltpu.sync_copy(table_hbm, table_v)

        plsc.subcore_barrier()

        def gather(i, b):
            return pltpu.make_async_copy(
                table_v.at[idx_v.at[pl.ds(i * CHUNK, CHUNK)]], rows[b], gsem[b]
            )

        def write(i, b):
            return pltpu.make_async_copy(
                rows[b], out_hbm.at[pl.ds(base + i * CHUNK, CHUNK)], wsem[b]
            )

        for b in range(GA):
            gather(b, b).start()

        for bb in range(NBUF):
            gather(bb, bb).wait()
            write(bb, bb).start()
            if bb + GA < NBUF:
                gather(bb + GA, bb + GA).start()
            else:
                write(bb + GA - NBUF, (bb + GA) % NBUF).wait()
                gather(bb + GA, (bb + GA) % NBUF).start()

        def body(grp, carry):
            for bb in range(NBUF):
                i = grp * NBUF + bb
                gather(i, bb).wait()
                write(i, bb).start()
                write(i - GA, (bb + GA) % NBUF).wait()
                gather(i + GA, (bb + GA) % NBUF).start()
            return carry

        lax.fori_loop(1, n_grps - 1, body, 0)

        for bb in range(NBUF):
            i = (n_grps - 1) * NBUF + bb
            gather(i, bb).wait()
            write(i, bb).start()
            if bb < GA:
                write(i - GA, (bb + GA) % NBUF).wait()
                gather(i + GA, (bb + GA) % NBUF).start()
        for bb in range(NBUF):
            write((n_grps - 1) * NBUF + bb, bb).wait()

    return k(idx_flat, table)


def kernel(cumulative_positions, position_embeddings):
    b, s = cumulative_positions.shape
    n_total = b * s
    n_rows = position_embeddings.shape[0]
    idx_flat = cumulative_positions.reshape(n_total).astype(jnp.int32)
    out = _sc_gather(idx_flat, position_embeddings, n_total, n_rows)
    return out.reshape(b, s, D_MODEL)
